# baseline (device time: 660472 ns/iter reference)
import jax
import jax.numpy as jnp
from jax import lax
from jax.experimental import pallas as pl
from jax.experimental.pallas import tpu as pltpu

N_DEV = 8


def kernel(x, w_mat):
    m_per, k = x.shape
    k2, n_per = w_mat.shape
    assert k == k2
    M = N_DEV * m_per

    def body(x_ref, w_ref, out_ref, comm_ref, amax_ref,
             send_sems, recv_sems, amax_send_sems, amax_recv_sems):
        my_pos = lax.axis_index("i")
        left = lax.rem(my_pos + (N_DEV - 1), N_DEV)
        right = lax.rem(my_pos + 1, N_DEV)

        barrier_sem = pltpu.get_barrier_semaphore()
        for nbr in (left, right):
            pl.semaphore_signal(
                barrier_sem, inc=1,
                device_id=(nbr,), device_id_type=pl.DeviceIdType.MESH,
            )
        pl.semaphore_wait(barrier_sem, 2)

        for h in range(N_DEV - 1):
            src = x_ref if h == 0 else comm_ref.at[h % 2]
            recv_slot = (h + 1) % 2
            rdma = pltpu.make_async_remote_copy(
                src_ref=src,
                dst_ref=comm_ref.at[recv_slot],
                send_sem=send_sems.at[h % 2],
                recv_sem=recv_sems.at[recv_slot],
                device_id=(right,),
                device_id_type=pl.DeviceIdType.MESH,
            )
            rdma.start()
            origin = lax.rem(my_pos + (N_DEV - h), N_DEV)
            chunk = x_ref[...] if h == 0 else comm_ref[h % 2]
            yb = jnp.dot(chunk, w_ref[...], preferred_element_type=jnp.float32)
            out_ref[pl.ds(origin * m_per, m_per), :] = jnp.maximum(yb, 0.0)
            rdma.wait()

        chunk = comm_ref[(N_DEV - 1) % 2]
        yb = jnp.dot(chunk, w_ref[...], preferred_element_type=jnp.float32)
        out_ref[pl.ds(right * m_per, m_per), :] = jnp.maximum(yb, 0.0)

        local_amax = jnp.max(out_ref[...])
        amax_ref[pl.ds(my_pos, 1)] = jnp.broadcast_to(local_amax, (1, 8, 128))
        amax_rdmas = []
        for d in range(1, N_DEV):
            tgt = lax.rem(my_pos + d, N_DEV)
            r = pltpu.make_async_remote_copy(
                src_ref=amax_ref.at[my_pos],
                dst_ref=amax_ref.at[my_pos],
                send_sem=amax_send_sems.at[d],
                recv_sem=amax_recv_sems.at[my_pos],
                device_id=(tgt,),
                device_id_type=pl.DeviceIdType.MESH,
            )
            r.start()
            amax_rdmas.append(r)
        for d in range(1, N_DEV):
            srcpos = lax.rem(my_pos + d, N_DEV)
            r = pltpu.make_async_remote_copy(
                src_ref=amax_ref.at[srcpos],
                dst_ref=amax_ref.at[srcpos],
                send_sem=amax_send_sems.at[0],
                recv_sem=amax_recv_sems.at[srcpos],
                device_id=(right,),
                device_id_type=pl.DeviceIdType.MESH,
            )
            r.wait_recv()
        for r in amax_rdmas:
            r.wait_send()

        gmax = jnp.max(amax_ref[...])
        scale = jnp.where(gmax > 0.0, gmax, 1.0) / 127.0
        q = jnp.clip(jnp.round(out_ref[...] / scale), -127.0, 127.0)
        out_ref[...] = q * scale

    return pl.pallas_call(
        body,
        out_shape=jax.ShapeDtypeStruct((M, n_per), jnp.float32),
        in_specs=[
            pl.BlockSpec(memory_space=pltpu.VMEM),
            pl.BlockSpec(memory_space=pltpu.VMEM),
        ],
        out_specs=pl.BlockSpec(memory_space=pltpu.VMEM),
        scratch_shapes=[
            pltpu.VMEM((2, m_per, k), jnp.float32),
            pltpu.VMEM((N_DEV, 8, 128), jnp.float32),
            pltpu.SemaphoreType.DMA((2,)),
            pltpu.SemaphoreType.DMA((2,)),
            pltpu.SemaphoreType.DMA((N_DEV,)),
            pltpu.SemaphoreType.DMA((N_DEV,)),
        ],
        compiler_params=pltpu.CompilerParams(collective_id=0),
    )(x, w_mat)


# device time: 347180 ns/iter; 1.9024x vs baseline; 1.9024x over previous
import jax
import jax.numpy as jnp
from jax import lax
from jax.experimental import pallas as pl
from jax.experimental.pallas import tpu as pltpu

N_DEV = 8


def kernel(x, w_mat):
    m_per, k = x.shape
    k2, n_per = w_mat.shape
    assert k == k2 and k % 2 == 0
    M = N_DEV * m_per
    kh = k // 2

    def body(x_ref, w_ref, out_ref, comm_a, comm_b, amax_ref,
             send_a, recv_a, send_b, recv_b, amax_send_sems, amax_recv_sems):
        my_pos = lax.axis_index("i")
        left = lax.rem(my_pos + (N_DEV - 1), N_DEV)
        right = lax.rem(my_pos + 1, N_DEV)

        barrier_sem = pltpu.get_barrier_semaphore()
        for nbr in (left, right):
            pl.semaphore_signal(
                barrier_sem, inc=1,
                device_id=(nbr,), device_id_type=pl.DeviceIdType.MESH,
            )
        pl.semaphore_wait(barrier_sem, 2)

        def block_acc(origin, yb, init):
            idx = pl.ds(origin * m_per, m_per)
            if init:
                out_ref[idx, :] = yb
            else:
                out_ref[idx, :] = out_ref[idx, :] + yb

        for h in range(N_DEV - 1):
            src_a = x_ref.at[:, pl.ds(0, kh)] if h == 0 else comm_a.at[h % 2]
            src_b = x_ref.at[:, pl.ds(kh, kh)] if h == 0 else comm_b.at[h % 2]
            recv_slot = (h + 1) % 2
            rdma_a = pltpu.make_async_remote_copy(
                src_ref=src_a,
                dst_ref=comm_a.at[recv_slot],
                send_sem=send_a.at[h % 2],
                recv_sem=recv_a.at[recv_slot],
                device_id=(right,),
                device_id_type=pl.DeviceIdType.MESH,
            )
            rdma_b = pltpu.make_async_remote_copy(
                src_ref=src_b,
                dst_ref=comm_b.at[recv_slot],
                send_sem=send_b.at[h % 2],
                recv_sem=recv_b.at[recv_slot],
                device_id=(left,),
                device_id_type=pl.DeviceIdType.MESH,
            )
            rdma_a.start()
            rdma_b.start()
            if h == 0:
                yb = jnp.dot(x_ref[...], w_ref[...],
                             preferred_element_type=jnp.float32)
                block_acc(my_pos, yb, init=True)
            else:
                ya = jnp.dot(comm_a[h % 2], w_ref[pl.ds(0, kh), :],
                             preferred_element_type=jnp.float32)
                block_acc(lax.rem(my_pos + (N_DEV - h), N_DEV), ya,
                          init=(h <= 4))
                yb = jnp.dot(comm_b[h % 2], w_ref[pl.ds(kh, kh), :],
                             preferred_element_type=jnp.float32)
                block_acc(lax.rem(my_pos + h, N_DEV), yb, init=(h <= 3))
            rdma_a.wait()
            rdma_b.wait()

        last = (N_DEV - 1) % 2
        ya = jnp.dot(comm_a[last], w_ref[pl.ds(0, kh), :],
                     preferred_element_type=jnp.float32)
        block_acc(right, ya, init=False)
        yb = jnp.dot(comm_b[last], w_ref[pl.ds(kh, kh), :],
                     preferred_element_type=jnp.float32)
        block_acc(left, yb, init=False)

        local_amax = jnp.maximum(jnp.max(out_ref[...]), 0.0)
        amax_ref[pl.ds(my_pos, 1)] = jnp.broadcast_to(local_amax, (1, 8, 128))
        amax_rdmas = []
        for d in range(1, N_DEV):
            tgt = lax.rem(my_pos + d, N_DEV)
            r = pltpu.make_async_remote_copy(
                src_ref=amax_ref.at[my_pos],
                dst_ref=amax_ref.at[my_pos],
                send_sem=amax_send_sems.at[d],
                recv_sem=amax_recv_sems.at[my_pos],
                device_id=(tgt,),
                device_id_type=pl.DeviceIdType.MESH,
            )
            r.start()
            amax_rdmas.append(r)
        for d in range(1, N_DEV):
            srcpos = lax.rem(my_pos + d, N_DEV)
            r = pltpu.make_async_remote_copy(
                src_ref=amax_ref.at[srcpos],
                dst_ref=amax_ref.at[srcpos],
                send_sem=amax_send_sems.at[0],
                recv_sem=amax_recv_sems.at[srcpos],
                device_id=(right,),
                device_id_type=pl.DeviceIdType.MESH,
            )
            r.wait_recv()
        for r in amax_rdmas:
            r.wait_send()

        gmax = jnp.max(amax_ref[...])
        scale = jnp.where(gmax > 0.0, gmax, 1.0) / 127.0
        y = jnp.maximum(out_ref[...], 0.0)
        q = jnp.clip(jnp.round(y / scale), -127.0, 127.0)
        out_ref[...] = q * scale

    return pl.pallas_call(
        body,
        out_shape=jax.ShapeDtypeStruct((M, n_per), jnp.float32),
        in_specs=[
            pl.BlockSpec(memory_space=pltpu.VMEM),
            pl.BlockSpec(memory_space=pltpu.VMEM),
        ],
        out_specs=pl.BlockSpec(memory_space=pltpu.VMEM),
        scratch_shapes=[
            pltpu.VMEM((2, m_per, kh), jnp.float32),
            pltpu.VMEM((2, m_per, kh), jnp.float32),
            pltpu.VMEM((N_DEV, 8, 128), jnp.float32),
            pltpu.SemaphoreType.DMA((2,)),
            pltpu.SemaphoreType.DMA((2,)),
            pltpu.SemaphoreType.DMA((2,)),
            pltpu.SemaphoreType.DMA((2,)),
            pltpu.SemaphoreType.DMA((N_DEV,)),
            pltpu.SemaphoreType.DMA((N_DEV,)),
        ],
        compiler_params=pltpu.CompilerParams(collective_id=0),
    )(x, w_mat)


# device time: 263806 ns/iter; 2.5036x vs baseline; 1.3160x over previous
import jax
import jax.numpy as jnp
from jax import lax
from jax.experimental import pallas as pl
from jax.experimental.pallas import tpu as pltpu

N_DEV = 8

ORDERS = ((1, 3, 4), (3, 4, 1), (4, 1, 3))


def _span(masks):
    vs = {0}
    for m in masks:
        vs |= {v ^ m for v in vs}
    return sorted(vs)


def kernel(x, w_mat):
    m_per, k = x.shape
    k2, n_per = w_mat.shape
    assert k == k2
    M = N_DEV * m_per

    lanes = k // 128
    l0 = (lanes + 2) // 3
    kps = (l0 * 128, l0 * 128, k - 2 * l0 * 128)
    koffs = (0, kps[0], kps[0] + kps[1])

    def body(x_ref, w_ref, out_ref, xg, stage, amax_ref,
             send_sems, recv_sems, copy_sems, amax_send_sems, amax_recv_sems):
        my_pos = lax.axis_index("i")

        barrier_sem = pltpu.get_barrier_semaphore()
        for m in (1, 3, 4):
            pl.semaphore_signal(
                barrier_sem, inc=1,
                device_id=(my_pos ^ m,), device_id_type=pl.DeviceIdType.MESH,
            )
        pl.semaphore_wait(barrier_sem, 3)

        for t in range(3):
            rdmas = []
            for s in range(3):
                m = ORDERS[s][t]
                partner = my_pos ^ m
                held = _span(ORDERS[s][:t])
                for v in held:
                    o = my_pos ^ v
                    if v == 0:
                        src = x_ref.at[:, pl.ds(koffs[s], kps[s])]
                    else:
                        src = xg.at[pl.ds(o * m_per, m_per),
                                    pl.ds(koffs[s], kps[s])]
                    r = pltpu.make_async_remote_copy(
                        src_ref=src,
                        dst_ref=xg.at[pl.ds(o * m_per, m_per),
                                      pl.ds(koffs[s], kps[s])],
                        send_sem=send_sems.at[s, v],
                        recv_sem=recv_sems.at[s, v ^ m],
                        device_id=(partner,),
                        device_id_type=pl.DeviceIdType.MESH,
                    )
                    r.start()
                    oin = my_pos ^ (v ^ m)
                    rin = pltpu.make_async_remote_copy(
                        src_ref=xg.at[pl.ds(oin * m_per, m_per),
                                      pl.ds(koffs[s], kps[s])],
                        dst_ref=xg.at[pl.ds(oin * m_per, m_per),
                                      pl.ds(koffs[s], kps[s])],
                        send_sem=send_sems.at[s, v],
                        recv_sem=recv_sems.at[s, v ^ m],
                        device_id=(partner,),
                        device_id_type=pl.DeviceIdType.MESH,
                    )
                    rdmas.append((r, rin))
            for r, rin in rdmas:
                r.wait_send()
                rin.wait_recv()

        def blk(v):
            return pl.ds((my_pos ^ v) * m_per, m_per)

        vs = list(range(1, N_DEV))

        def stage_copy(v, slot):
            return pltpu.make_async_copy(
                xg.at[blk(v), :], stage.at[slot], copy_sems.at[slot])

        stage_copy(vs[0], 0).start()
        y0 = jnp.dot(x_ref[...], w_ref[...], preferred_element_type=jnp.float32)
        out_ref[blk(0), :] = y0
        for j, v in enumerate(vs):
            if j + 1 < len(vs):
                stage_copy(vs[j + 1], (j + 1) % 2).start()
            stage_copy(v, j % 2).wait()
            yb = jnp.dot(stage[j % 2], w_ref[...],
                         preferred_element_type=jnp.float32)
            out_ref[blk(v), :] = yb

        local_amax = jnp.maximum(jnp.max(out_ref[...]), 0.0)
        amax_ref[pl.ds(my_pos, 1)] = jnp.broadcast_to(local_amax, (1, 8, 128))
        amax_rdmas = []
        for d in range(1, N_DEV):
            tgt = lax.rem(my_pos + d, N_DEV)
            r = pltpu.make_async_remote_copy(
                src_ref=amax_ref.at[my_pos],
                dst_ref=amax_ref.at[my_pos],
                send_sem=amax_send_sems.at[d],
                recv_sem=amax_recv_sems.at[my_pos],
                device_id=(tgt,),
                device_id_type=pl.DeviceIdType.MESH,
            )
            r.start()
            amax_rdmas.append(r)
        for d in range(1, N_DEV):
            srcpos = lax.rem(my_pos + d, N_DEV)
            r = pltpu.make_async_remote_copy(
                src_ref=amax_ref.at[srcpos],
                dst_ref=amax_ref.at[srcpos],
                send_sem=amax_send_sems.at[0],
                recv_sem=amax_recv_sems.at[srcpos],
                device_id=(my_pos ^ 1,),
                device_id_type=pl.DeviceIdType.MESH,
            )
            r.wait_recv()
        for r in amax_rdmas:
            r.wait_send()

        gmax = jnp.max(amax_ref[...])
        scale = jnp.where(gmax > 0.0, gmax, 1.0) / 127.0
        yr = jnp.maximum(out_ref[...], 0.0)
        q = jnp.clip(jnp.round(yr / scale), -127.0, 127.0)
        out_ref[...] = q * scale

    out, _ = pl.pallas_call(
        body,
        out_shape=[
            jax.ShapeDtypeStruct((M, n_per), jnp.float32),
            jax.ShapeDtypeStruct((M, k), jnp.float32),
        ],
        in_specs=[
            pl.BlockSpec(memory_space=pltpu.VMEM),
            pl.BlockSpec(memory_space=pltpu.VMEM),
        ],
        out_specs=[
            pl.BlockSpec(memory_space=pltpu.VMEM),
            pl.BlockSpec(memory_space=pltpu.HBM),
        ],
        scratch_shapes=[
            pltpu.VMEM((2, m_per, k), jnp.float32),
            pltpu.VMEM((N_DEV, 8, 128), jnp.float32),
            pltpu.SemaphoreType.DMA((3, N_DEV)),
            pltpu.SemaphoreType.DMA((3, N_DEV)),
            pltpu.SemaphoreType.DMA((2,)),
            pltpu.SemaphoreType.DMA((N_DEV,)),
            pltpu.SemaphoreType.DMA((N_DEV,)),
        ],
        compiler_params=pltpu.CompilerParams(collective_id=0),
    )(x, w_mat)
    return out


# device time: 258101 ns/iter; 2.5590x vs baseline; 1.0221x over previous
import jax
import jax.numpy as jnp
from jax import lax
from jax.experimental import pallas as pl
from jax.experimental.pallas import tpu as pltpu

N_DEV = 8

ORDERS = ((1, 3, 4), (3, 4, 1), (4, 1, 3))


def _span(masks):
    vs = {0}
    for m in masks:
        vs |= {v ^ m for v in vs}
    return sorted(vs)


def _held(s, t):
    return _span(ORDERS[s][:t])


def _arrivals(s, t):
    m = ORDERS[s][t]
    return sorted(v ^ m for v in _held(s, t))


def kernel(x, w_mat):
    m_per, k = x.shape
    k2, n_per = w_mat.shape
    assert k == k2
    M = N_DEV * m_per

    lanes = k // 128
    l0 = (lanes + 2) // 3
    kps = (l0 * 128, l0 * 128, k - 2 * l0 * 128)
    koffs = (0, kps[0], kps[0] + kps[1])
    kp_max = max(kps)

    def body(x_ref, w_ref, out_ref, xg, stage, amax_ref,
             send_sems, recv_sems, copy_sems, amax_send_sems, amax_recv_sems):
        my_pos = lax.axis_index("i")

        barrier_sem = pltpu.get_barrier_semaphore()
        for m in (1, 3, 4):
            pl.semaphore_signal(
                barrier_sem, inc=1,
                device_id=(my_pos ^ m,), device_id_type=pl.DeviceIdType.MESH,
            )
        pl.semaphore_wait(barrier_sem, 3)

        touched = set()

        def acc(vp, yb):
            o = my_pos ^ vp
            idx = pl.ds(o * m_per, m_per)
            if vp in touched:
                out_ref[idx, :] = out_ref[idx, :] + yb
            else:
                out_ref[idx, :] = yb
                touched.add(vp)

        def part_src(s, v):
            if v == 0:
                return x_ref.at[:, pl.ds(koffs[s], kps[s])]
            o = my_pos ^ v
            return xg.at[pl.ds(o * m_per, m_per), pl.ds(koffs[s], kps[s])]

        def issue_stage(t):
            rdmas = []
            for s in range(3):
                m = ORDERS[s][t]
                partner = my_pos ^ m
                arr = _arrivals(s, t)
                for v in _held(s, t):
                    vin = v ^ m
                    dst_out = xg.at[pl.ds((my_pos ^ v) * m_per, m_per),
                                    pl.ds(koffs[s], kps[s])]
                    dst_in = xg.at[pl.ds((my_pos ^ vin) * m_per, m_per),
                                   pl.ds(koffs[s], kps[s])]
                    r = pltpu.make_async_remote_copy(
                        src_ref=part_src(s, v),
                        dst_ref=dst_out,
                        send_sem=send_sems.at[s, v],
                        recv_sem=recv_sems.at[s, vin],
                        device_id=(partner,),
                        device_id_type=pl.DeviceIdType.MESH,
                    )
                    r.start()
                    rin = pltpu.make_async_remote_copy(
                        src_ref=dst_in,
                        dst_ref=dst_in,
                        send_sem=send_sems.at[s, v],
                        recv_sem=recv_sems.at[s, vin],
                        device_id=(partner,),
                        device_id_type=pl.DeviceIdType.MESH,
                    )
                    rdmas.append((r, rin))
            return rdmas

        def wait_stage(rdmas):
            for r, rin in rdmas:
                r.wait_send()
                rin.wait_recv()

        def staged_gemm(parts):
            def copy_desc(part, slot):
                s, vp = part
                return pltpu.make_async_copy(
                    xg.at[pl.ds((my_pos ^ vp) * m_per, m_per),
                          pl.ds(koffs[s], kps[s])],
                    stage.at[slot, :, pl.ds(0, kps[s])],
                    copy_sems.at[slot],
                )
            copy_desc(parts[0], 0).start()
            for j, part in enumerate(parts):
                if j + 1 < len(parts):
                    copy_desc(parts[j + 1], (j + 1) % 2).start()
                copy_desc(part, j % 2).wait()
                s, vp = part
                yb = jnp.dot(stage[j % 2, :, pl.ds(0, kps[s])],
                             w_ref[pl.ds(koffs[s], kps[s]), :],
                             preferred_element_type=jnp.float32)
                acc(vp, yb)

        rd = issue_stage(0)
        y0 = jnp.dot(x_ref[...], w_ref[...], preferred_element_type=jnp.float32)
        acc(0, y0)
        wait_stage(rd)

        rd = issue_stage(1)
        staged_gemm([(s, vp) for s in range(3) for vp in _arrivals(s, 0)])
        wait_stage(rd)

        rd = issue_stage(2)
        staged_gemm([(s, vp) for s in range(3) for vp in _arrivals(s, 1)])
        wait_stage(rd)

        staged_gemm([(s, vp) for s in range(3) for vp in _arrivals(s, 2)])
        assert sorted(touched) == list(range(N_DEV))

        local_amax = jnp.maximum(jnp.max(out_ref[...]), 0.0)
        amax_ref[pl.ds(my_pos, 1)] = jnp.broadcast_to(local_amax, (1, 8, 128))
        amax_rdmas = []
        for d in range(1, N_DEV):
            tgt = lax.rem(my_pos + d, N_DEV)
            r = pltpu.make_async_remote_copy(
                src_ref=amax_ref.at[my_pos],
                dst_ref=amax_ref.at[my_pos],
                send_sem=amax_send_sems.at[d],
                recv_sem=amax_recv_sems.at[my_pos],
                device_id=(tgt,),
                device_id_type=pl.DeviceIdType.MESH,
            )
            r.start()
            amax_rdmas.append(r)
        for d in range(1, N_DEV):
            srcpos = lax.rem(my_pos + d, N_DEV)
            r = pltpu.make_async_remote_copy(
                src_ref=amax_ref.at[srcpos],
                dst_ref=amax_ref.at[srcpos],
                send_sem=amax_send_sems.at[0],
                recv_sem=amax_recv_sems.at[srcpos],
                device_id=(my_pos ^ 1,),
                device_id_type=pl.DeviceIdType.MESH,
            )
            r.wait_recv()
        for r in amax_rdmas:
            r.wait_send()

        gmax = jnp.max(amax_ref[...])
        scale = jnp.where(gmax > 0.0, gmax, 1.0) / 127.0
        yr = jnp.maximum(out_ref[...], 0.0)
        q = jnp.clip(jnp.round(yr / scale), -127.0, 127.0)
        out_ref[...] = q * scale

    out, _ = pl.pallas_call(
        body,
        out_shape=[
            jax.ShapeDtypeStruct((M, n_per), jnp.float32),
            jax.ShapeDtypeStruct((M, k), jnp.float32),
        ],
        in_specs=[
            pl.BlockSpec(memory_space=pltpu.VMEM),
            pl.BlockSpec(memory_space=pltpu.VMEM),
        ],
        out_specs=[
            pl.BlockSpec(memory_space=pltpu.VMEM),
            pl.BlockSpec(memory_space=pltpu.HBM),
        ],
        scratch_shapes=[
            pltpu.VMEM((2, m_per, kp_max), jnp.float32),
            pltpu.VMEM((N_DEV, 8, 128), jnp.float32),
            pltpu.SemaphoreType.DMA((3, N_DEV)),
            pltpu.SemaphoreType.DMA((3, N_DEV)),
            pltpu.SemaphoreType.DMA((2,)),
            pltpu.SemaphoreType.DMA((N_DEV,)),
            pltpu.SemaphoreType.DMA((N_DEV,)),
        ],
        compiler_params=pltpu.CompilerParams(
            collective_id=0,
            vmem_limit_bytes=100 * 1024 * 1024,
        ),
    )(x, w_mat)
    return out


# device time: 247137 ns/iter; 2.6725x vs baseline; 1.0444x over previous
import jax
import jax.numpy as jnp
from jax import lax
from jax.experimental import pallas as pl
from jax.experimental.pallas import tpu as pltpu

N_DEV = 8

ORDERS = ((1, 3, 4), (3, 4, 1), (4, 1, 3))


def _span(masks):
    vs = {0}
    for m in masks:
        vs |= {v ^ m for v in vs}
    return sorted(vs)


def _held(s, t):
    return _span(ORDERS[s][:t])


def _arrivals(s, t):
    m = ORDERS[s][t]
    return sorted(v ^ m for v in _held(s, t))


def kernel(x, w_mat):
    m_per, k = x.shape
    k2, n_per = w_mat.shape
    assert k == k2
    M = N_DEV * m_per

    lanes = k // 128
    l0 = (lanes + 2) // 3
    kps = (l0 * 128, l0 * 128, k - 2 * l0 * 128)
    koffs = (0, kps[0], kps[0] + kps[1])
    kp_max = max(kps)

    def body(x_ref, w_ref, out_ref, xg, vrecv, stage, amax_ref,
             send_sems, recv_sems, copy_sems, amax_send_sems, amax_recv_sems):
        my_pos = lax.axis_index("i")

        barrier_sem = pltpu.get_barrier_semaphore()
        for m in (1, 3, 4):
            pl.semaphore_signal(
                barrier_sem, inc=1,
                device_id=(my_pos ^ m,), device_id_type=pl.DeviceIdType.MESH,
            )
        pl.semaphore_wait(barrier_sem, 3)

        touched = set()

        def acc(vp, yb):
            o = my_pos ^ vp
            idx = pl.ds(o * m_per, m_per)
            if vp in touched:
                out_ref[idx, :] = out_ref[idx, :] + yb
            else:
                out_ref[idx, :] = yb
                touched.add(vp)

        def part_src(s, v):
            if v == 0:
                return x_ref.at[:, pl.ds(koffs[s], kps[s])]
            o = my_pos ^ v
            return xg.at[pl.ds(o * m_per, m_per), pl.ds(koffs[s], kps[s])]

        def issue_stage(t):
            rdmas = []
            for s in range(3):
                m = ORDERS[s][t]
                partner = my_pos ^ m
                arr = _arrivals(s, t)
                for v in _held(s, t):
                    vin = v ^ m
                    if t < 2:
                        dst_out = xg.at[pl.ds((my_pos ^ v) * m_per, m_per),
                                        pl.ds(koffs[s], kps[s])]
                        dst_in = xg.at[pl.ds((my_pos ^ vin) * m_per, m_per),
                                       pl.ds(koffs[s], kps[s])]
                    else:
                        dst_out = vrecv.at[s, arr.index(vin), :,
                                           pl.ds(0, kps[s])]
                        dst_in = dst_out
                    r = pltpu.make_async_remote_copy(
                        src_ref=part_src(s, v),
                        dst_ref=dst_out,
                        send_sem=send_sems.at[s, v],
                        recv_sem=recv_sems.at[s, vin],
                        device_id=(partner,),
                        device_id_type=pl.DeviceIdType.MESH,
                    )
                    r.start()
                    rin = pltpu.make_async_remote_copy(
                        src_ref=dst_in,
                        dst_ref=dst_in,
                        send_sem=send_sems.at[s, v],
                        recv_sem=recv_sems.at[s, vin],
                        device_id=(partner,),
                        device_id_type=pl.DeviceIdType.MESH,
                    )
                    rdmas.append((r, rin))
            return rdmas

        def wait_stage(rdmas):
            for r, rin in rdmas:
                r.wait_send()
                rin.wait_recv()

        def staged_gemm(parts):
            def copy_desc(part, slot):
                s, vp = part
                return pltpu.make_async_copy(
                    xg.at[pl.ds((my_pos ^ vp) * m_per, m_per),
                          pl.ds(koffs[s], kps[s])],
                    stage.at[slot, :, pl.ds(0, kps[s])],
                    copy_sems.at[slot],
                )
            copy_desc(parts[0], 0).start()
            for j, part in enumerate(parts):
                if j + 1 < len(parts):
                    copy_desc(parts[j + 1], (j + 1) % 2).start()
                copy_desc(part, j % 2).wait()
                s, vp = part
                yb = jnp.dot(stage[j % 2, :, pl.ds(0, kps[s])],
                             w_ref[pl.ds(koffs[s], kps[s]), :],
                             preferred_element_type=jnp.float32)
                acc(vp, yb)

        rd = issue_stage(0)
        y0 = jnp.dot(x_ref[...], w_ref[...], preferred_element_type=jnp.float32)
        acc(0, y0)
        wait_stage(rd)

        rd = issue_stage(1)
        staged_gemm([(s, vp) for s in range(3) for vp in _arrivals(s, 0)])
        wait_stage(rd)

        rd = issue_stage(2)
        staged_gemm([(s, vp) for s in range(3) for vp in _arrivals(s, 1)])
        wait_stage(rd)

        for s in range(3):
            for slot, vp in enumerate(_arrivals(s, 2)):
                yb = jnp.dot(vrecv[s, slot, :, pl.ds(0, kps[s])],
                             w_ref[pl.ds(koffs[s], kps[s]), :],
                             preferred_element_type=jnp.float32)
                acc(vp, yb)
        assert sorted(touched) == list(range(N_DEV))

        local_amax = jnp.maximum(jnp.max(out_ref[...]), 0.0)
        amax_ref[pl.ds(my_pos, 1)] = jnp.broadcast_to(local_amax, (1, 8, 128))
        amax_rdmas = []
        for d in range(1, N_DEV):
            tgt = lax.rem(my_pos + d, N_DEV)
            r = pltpu.make_async_remote_copy(
                src_ref=amax_ref.at[my_pos],
                dst_ref=amax_ref.at[my_pos],
                send_sem=amax_send_sems.at[d],
                recv_sem=amax_recv_sems.at[my_pos],
                device_id=(tgt,),
                device_id_type=pl.DeviceIdType.MESH,
            )
            r.start()
            amax_rdmas.append(r)
        for d in range(1, N_DEV):
            srcpos = lax.rem(my_pos + d, N_DEV)
            r = pltpu.make_async_remote_copy(
                src_ref=amax_ref.at[srcpos],
                dst_ref=amax_ref.at[srcpos],
                send_sem=amax_send_sems.at[0],
                recv_sem=amax_recv_sems.at[srcpos],
                device_id=(my_pos ^ 1,),
                device_id_type=pl.DeviceIdType.MESH,
            )
            r.wait_recv()
        for r in amax_rdmas:
            r.wait_send()

        gmax = jnp.max(amax_ref[...])
        scale = jnp.where(gmax > 0.0, gmax, 1.0) / 127.0
        yr = jnp.maximum(out_ref[...], 0.0)
        q = jnp.clip(jnp.round(yr / scale), -127.0, 127.0)
        out_ref[...] = q * scale

    out, _ = pl.pallas_call(
        body,
        out_shape=[
            jax.ShapeDtypeStruct((M, n_per), jnp.float32),
            jax.ShapeDtypeStruct((M, k), jnp.float32),
        ],
        in_specs=[
            pl.BlockSpec(memory_space=pltpu.VMEM),
            pl.BlockSpec(memory_space=pltpu.VMEM),
        ],
        out_specs=[
            pl.BlockSpec(memory_space=pltpu.VMEM),
            pl.BlockSpec(memory_space=pltpu.HBM),
        ],
        scratch_shapes=[
            pltpu.VMEM((3, 4, m_per, kp_max), jnp.float32),
            pltpu.VMEM((2, m_per, kp_max), jnp.float32),
            pltpu.VMEM((N_DEV, 8, 128), jnp.float32),
            pltpu.SemaphoreType.DMA((3, N_DEV)),
            pltpu.SemaphoreType.DMA((3, N_DEV)),
            pltpu.SemaphoreType.DMA((2,)),
            pltpu.SemaphoreType.DMA((N_DEV,)),
            pltpu.SemaphoreType.DMA((N_DEV,)),
        ],
        compiler_params=pltpu.CompilerParams(
            collective_id=0,
            vmem_limit_bytes=100 * 1024 * 1024,
        ),
    )(x, w_mat)
    return out


# device time: 245238 ns/iter; 2.6932x vs baseline; 1.0077x over previous
import jax
import jax.numpy as jnp
from jax import lax
from jax.experimental import pallas as pl
from jax.experimental.pallas import tpu as pltpu

N_DEV = 8

ORDERS = ((1, 3, 4), (3, 4, 1), (4, 1, 3))


def _span(masks):
    vs = {0}
    for m in masks:
        vs |= {v ^ m for v in vs}
    return sorted(vs)


def _held(s, t):
    return _span(ORDERS[s][:t])


def _arrivals(s, t):
    m = ORDERS[s][t]
    return sorted(v ^ m for v in _held(s, t))


def kernel(x, w_mat):
    m_per, k = x.shape
    k2, n_per = w_mat.shape
    assert k == k2
    M = N_DEV * m_per

    lanes = k // 128
    l0 = (lanes + 2) // 3
    kps = (l0 * 128, l0 * 128, k - 2 * l0 * 128)
    koffs = (0, kps[0], kps[0] + kps[1])
    kp_max = max(kps)

    def body(x_ref, w_ref, out_ref, xg, vrecv, stage, amax_ref,
             send_sems, recv_sems, copy_sems, amax_send_sems, amax_recv_sems):
        my_pos = lax.axis_index("i")

        barrier_sem = pltpu.get_barrier_semaphore()
        for m in (1, 3, 4):
            pl.semaphore_signal(
                barrier_sem, inc=1,
                device_id=(my_pos ^ m,), device_id_type=pl.DeviceIdType.MESH,
            )
        pl.semaphore_wait(barrier_sem, 3)

        touched = set()

        def acc(vp, yb):
            o = my_pos ^ vp
            idx = pl.ds(o * m_per, m_per)
            if vp in touched:
                out_ref[idx, :] = out_ref[idx, :] + yb
            else:
                out_ref[idx, :] = yb
                touched.add(vp)

        def part_src(s, v):
            if v == 0:
                return x_ref.at[:, pl.ds(koffs[s], kps[s])]
            o = my_pos ^ v
            return xg.at[pl.ds(o * m_per, m_per), pl.ds(koffs[s], kps[s])]

        def issue_stage(t):
            rdmas = []
            for s in range(3):
                m = ORDERS[s][t]
                partner = my_pos ^ m
                arr = _arrivals(s, t)
                for v in _held(s, t):
                    vin = v ^ m
                    if t < 2:
                        dst_out = xg.at[pl.ds((my_pos ^ v) * m_per, m_per),
                                        pl.ds(koffs[s], kps[s])]
                        dst_in = xg.at[pl.ds((my_pos ^ vin) * m_per, m_per),
                                       pl.ds(koffs[s], kps[s])]
                    else:
                        dst_out = vrecv.at[s, arr.index(vin), :,
                                           pl.ds(0, kps[s])]
                        dst_in = dst_out
                    r = pltpu.make_async_remote_copy(
                        src_ref=part_src(s, v),
                        dst_ref=dst_out,
                        send_sem=send_sems.at[s, v],
                        recv_sem=recv_sems.at[s, vin],
                        device_id=(partner,),
                        device_id_type=pl.DeviceIdType.MESH,
                    )
                    r.start()
                    rin = pltpu.make_async_remote_copy(
                        src_ref=dst_in,
                        dst_ref=dst_in,
                        send_sem=send_sems.at[s, v],
                        recv_sem=recv_sems.at[s, vin],
                        device_id=(partner,),
                        device_id_type=pl.DeviceIdType.MESH,
                    )
                    rdmas.append((r, rin))
            return rdmas

        def wait_stage(rdmas):
            for r, rin in rdmas:
                r.wait_send()
                rin.wait_recv()

        def staged_gemm(parts):
            def copy_desc(part, slot):
                s, vp = part
                return pltpu.make_async_copy(
                    xg.at[pl.ds((my_pos ^ vp) * m_per, m_per),
                          pl.ds(koffs[s], kps[s])],
                    stage.at[slot, :, pl.ds(0, kps[s])],
                    copy_sems.at[slot],
                )
            copy_desc(parts[0], 0).start()
            for j, part in enumerate(parts):
                if j + 1 < len(parts):
                    copy_desc(parts[j + 1], (j + 1) % 2).start()
                copy_desc(part, j % 2).wait()
                s, vp = part
                yb = jnp.dot(stage[j % 2, :, pl.ds(0, kps[s])],
                             w_ref[pl.ds(koffs[s], kps[s]), :],
                             preferred_element_type=jnp.float32)
                acc(vp, yb)

        rd = issue_stage(0)
        y0 = jnp.dot(x_ref[...], w_ref[...], preferred_element_type=jnp.float32)
        acc(0, y0)
        wait_stage(rd)

        rd = issue_stage(1)
        staged_gemm([(s, vp) for s in range(3) for vp in _arrivals(s, 0)])
        wait_stage(rd)

        rd2 = issue_stage(2)
        staged_gemm([(s, vp) for s in range(3) for vp in _arrivals(s, 1)])

        for j in range(4):
            for s in range(3):
                _, rin = rd2[s * 4 + j]
                rin.wait_recv()
                vin = _held(s, 2)[j] ^ ORDERS[s][2]
                slot = _arrivals(s, 2).index(vin)
                yb = jnp.dot(vrecv[s, slot, :, pl.ds(0, kps[s])],
                             w_ref[pl.ds(koffs[s], kps[s]), :],
                             preferred_element_type=jnp.float32)
                acc(vin, yb)
        for r, _ in rd2:
            r.wait_send()
        assert sorted(touched) == list(range(N_DEV))

        local_amax = jnp.maximum(jnp.max(out_ref[...]), 0.0)
        amax_ref[pl.ds(my_pos, 1)] = jnp.broadcast_to(local_amax, (1, 8, 128))
        amax_rdmas = []
        for d in range(1, N_DEV):
            tgt = lax.rem(my_pos + d, N_DEV)
            r = pltpu.make_async_remote_copy(
                src_ref=amax_ref.at[my_pos],
                dst_ref=amax_ref.at[my_pos],
                send_sem=amax_send_sems.at[d],
                recv_sem=amax_recv_sems.at[my_pos],
                device_id=(tgt,),
                device_id_type=pl.DeviceIdType.MESH,
            )
            r.start()
            amax_rdmas.append(r)
        for d in range(1, N_DEV):
            srcpos = lax.rem(my_pos + d, N_DEV)
            r = pltpu.make_async_remote_copy(
                src_ref=amax_ref.at[srcpos],
                dst_ref=amax_ref.at[srcpos],
                send_sem=amax_send_sems.at[0],
                recv_sem=amax_recv_sems.at[srcpos],
                device_id=(my_pos ^ 1,),
                device_id_type=pl.DeviceIdType.MESH,
            )
            r.wait_recv()
        for r in amax_rdmas:
            r.wait_send()

        gmax = jnp.max(amax_ref[...])
        scale = jnp.where(gmax > 0.0, gmax, 1.0) / 127.0
        yr = jnp.maximum(out_ref[...], 0.0)
        q = jnp.clip(jnp.round(yr / scale), -127.0, 127.0)
        out_ref[...] = q * scale

    out, _ = pl.pallas_call(
        body,
        out_shape=[
            jax.ShapeDtypeStruct((M, n_per), jnp.float32),
            jax.ShapeDtypeStruct((M, k), jnp.float32),
        ],
        in_specs=[
            pl.BlockSpec(memory_space=pltpu.VMEM),
            pl.BlockSpec(memory_space=pltpu.VMEM),
        ],
        out_specs=[
            pl.BlockSpec(memory_space=pltpu.VMEM),
            pl.BlockSpec(memory_space=pltpu.HBM),
        ],
        scratch_shapes=[
            pltpu.VMEM((3, 4, m_per, kp_max), jnp.float32),
            pltpu.VMEM((2, m_per, kp_max), jnp.float32),
            pltpu.VMEM((N_DEV, 8, 128), jnp.float32),
            pltpu.SemaphoreType.DMA((3, N_DEV)),
            pltpu.SemaphoreType.DMA((3, N_DEV)),
            pltpu.SemaphoreType.DMA((2,)),
            pltpu.SemaphoreType.DMA((N_DEV,)),
            pltpu.SemaphoreType.DMA((N_DEV,)),
        ],
        compiler_params=pltpu.CompilerParams(
            collective_id=0,
            vmem_limit_bytes=100 * 1024 * 1024,
        ),
    )(x, w_mat)
    return out


# device time: 241788 ns/iter; 2.7316x vs baseline; 1.0143x over previous
import jax
import jax.numpy as jnp
from jax import lax
from jax.experimental import pallas as pl
from jax.experimental.pallas import tpu as pltpu

N_DEV = 8

ORDERS = ((1, 3, 4), (3, 4, 1), (4, 1, 3))


def _span(masks):
    vs = {0}
    for m in masks:
        vs |= {v ^ m for v in vs}
    return sorted(vs)


def _held(s, t):
    return _span(ORDERS[s][:t])


def _arrivals(s, t):
    m = ORDERS[s][t]
    return sorted(v ^ m for v in _held(s, t))


def kernel(x, w_mat):
    m_per, k = x.shape
    k2, n_per = w_mat.shape
    assert k == k2
    M = N_DEV * m_per

    lanes = k // 128
    l0 = (lanes + 2) // 3
    kps = (l0 * 128, l0 * 128, k - 2 * l0 * 128)
    koffs = (0, kps[0], kps[0] + kps[1])
    kp_max = max(kps)

    def body(x_ref, w_ref, out_ref, xg, vrecv, stage, amax_ref,
             send_sems, recv_sems, copy_sems, amax_send_sems, amax_recv_sems):
        my_pos = lax.axis_index("i")

        barrier_sem = pltpu.get_barrier_semaphore()
        for m in (1, 3, 4):
            pl.semaphore_signal(
                barrier_sem, inc=1,
                device_id=(my_pos ^ m,), device_id_type=pl.DeviceIdType.MESH,
            )
        pl.semaphore_wait(barrier_sem, 3)

        touched = {}
        amax_run = [jnp.float32(0.0)]

        def acc(vp, yb):
            o = my_pos ^ vp
            idx = pl.ds(o * m_per, m_per)
            seen = touched.get(vp, 0)
            val = yb if seen == 0 else out_ref[idx, :] + yb
            out_ref[idx, :] = val
            touched[vp] = seen + 1
            if touched[vp] == (1 if vp == 0 else 3):
                amax_run[0] = jnp.maximum(amax_run[0], jnp.max(val))

        def part_src(s, v):
            if v == 0:
                return x_ref.at[:, pl.ds(koffs[s], kps[s])]
            o = my_pos ^ v
            return xg.at[pl.ds(o * m_per, m_per), pl.ds(koffs[s], kps[s])]

        def issue_subset(t, vs_of):
            rdmas = []
            for s in range(3):
                m = ORDERS[s][t]
                partner = my_pos ^ m
                arr = _arrivals(s, t)
                for v in vs_of(s):
                    vin = v ^ m
                    if t < 2:
                        dst_out = xg.at[pl.ds((my_pos ^ v) * m_per, m_per),
                                        pl.ds(koffs[s], kps[s])]
                        dst_in = xg.at[pl.ds((my_pos ^ vin) * m_per, m_per),
                                       pl.ds(koffs[s], kps[s])]
                    else:
                        dst_out = vrecv.at[s, arr.index(vin), :,
                                           pl.ds(0, kps[s])]
                        dst_in = dst_out
                    r = pltpu.make_async_remote_copy(
                        src_ref=part_src(s, v),
                        dst_ref=dst_out,
                        send_sem=send_sems.at[s, v],
                        recv_sem=recv_sems.at[s, vin],
                        device_id=(partner,),
                        device_id_type=pl.DeviceIdType.MESH,
                    )
                    r.start()
                    rin = pltpu.make_async_remote_copy(
                        src_ref=dst_in,
                        dst_ref=dst_in,
                        send_sem=send_sems.at[s, v],
                        recv_sem=recv_sems.at[s, vin],
                        device_id=(partner,),
                        device_id_type=pl.DeviceIdType.MESH,
                    )
                    rdmas.append((r, rin))
            return rdmas

        def wait_sends(rdmas):
            for r, _ in rdmas:
                r.wait_send()

        def wait_recvs(rdmas):
            for _, rin in rdmas:
                rin.wait_recv()

        def staged_gemm(parts):
            def copy_desc(part, slot):
                s, vp = part
                return pltpu.make_async_copy(
                    xg.at[pl.ds((my_pos ^ vp) * m_per, m_per),
                          pl.ds(koffs[s], kps[s])],
                    stage.at[slot, :, pl.ds(0, kps[s])],
                    copy_sems.at[slot],
                )
            copy_desc(parts[0], 0).start()
            for j, part in enumerate(parts):
                if j + 1 < len(parts):
                    copy_desc(parts[j + 1], (j + 1) % 2).start()
                copy_desc(part, j % 2).wait()
                s, vp = part
                yb = jnp.dot(stage[j % 2, :, pl.ds(0, kps[s])],
                             w_ref[pl.ds(koffs[s], kps[s]), :],
                             preferred_element_type=jnp.float32)
                acc(vp, yb)


        rd0 = issue_subset(0, lambda s: [0])
        y0 = jnp.dot(x_ref[...], w_ref[...], preferred_element_type=jnp.float32)
        acc(0, y0)
        wait_sends(rd0)
        rd1o = issue_subset(1, lambda s: _held(s, 0))
        wait_recvs(rd0)
        rd1n = issue_subset(1, lambda s: _arrivals(s, 0))
        rd1 = rd1o + rd1n
        staged_gemm([(s, vp) for s in range(3) for vp in _arrivals(s, 0)])
        wait_sends(rd1)
        rd2o = issue_subset(2, lambda s: _held(s, 1))
        wait_recvs(rd1)
        rd2n = issue_subset(2, lambda s: _arrivals(s, 1))
        staged_gemm([(s, vp) for s in range(3) for vp in _arrivals(s, 1)])

        pair = {}
        for i, (s, v) in enumerate(
                [(s, v) for s in range(3) for v in _held(s, 1)]):
            pair[(s, v)] = rd2o[i]
        for i, (s, v) in enumerate(
                [(s, v) for s in range(3) for v in _arrivals(s, 1)]):
            pair[(s, v)] = rd2n[i]
        issue_order = {s: _held(s, 1) + _arrivals(s, 1) for s in range(3)}
        for j in range(4):
            for s in range(3):
                v = issue_order[s][j]
                _, rin = pair[(s, v)]
                rin.wait_recv()
                vin = v ^ ORDERS[s][2]
                slot = _arrivals(s, 2).index(vin)
                yb = jnp.dot(vrecv[s, slot, :, pl.ds(0, kps[s])],
                             w_ref[pl.ds(koffs[s], kps[s]), :],
                             preferred_element_type=jnp.float32)
                acc(vin, yb)
        wait_sends(rd2o + rd2n)
        assert all(touched[vp] == (1 if vp == 0 else 3)
                   for vp in range(N_DEV))

        local_amax = amax_run[0]
        amax_ref[pl.ds(my_pos, 1)] = jnp.broadcast_to(local_amax, (1, 8, 128))
        amax_rdmas = []
        for d in range(1, N_DEV):
            tgt = lax.rem(my_pos + d, N_DEV)
            r = pltpu.make_async_remote_copy(
                src_ref=amax_ref.at[my_pos],
                dst_ref=amax_ref.at[my_pos],
                send_sem=amax_send_sems.at[d],
                recv_sem=amax_recv_sems.at[my_pos],
                device_id=(tgt,),
                device_id_type=pl.DeviceIdType.MESH,
            )
            r.start()
            amax_rdmas.append(r)
        for d in range(1, N_DEV):
            srcpos = lax.rem(my_pos + d, N_DEV)
            r = pltpu.make_async_remote_copy(
                src_ref=amax_ref.at[srcpos],
                dst_ref=amax_ref.at[srcpos],
                send_sem=amax_send_sems.at[0],
                recv_sem=amax_recv_sems.at[srcpos],
                device_id=(my_pos ^ 1,),
                device_id_type=pl.DeviceIdType.MESH,
            )
            r.wait_recv()
        for r in amax_rdmas:
            r.wait_send()

        gmax = jnp.max(amax_ref[...])
        scale = jnp.where(gmax > 0.0, gmax, 1.0) / 127.0
        yr = jnp.maximum(out_ref[...], 0.0)
        q = jnp.clip(jnp.round(yr / scale), -127.0, 127.0)
        out_ref[...] = q * scale

    out, _ = pl.pallas_call(
        body,
        out_shape=[
            jax.ShapeDtypeStruct((M, n_per), jnp.float32),
            jax.ShapeDtypeStruct((M, k), jnp.float32),
        ],
        in_specs=[
            pl.BlockSpec(memory_space=pltpu.VMEM),
            pl.BlockSpec(memory_space=pltpu.VMEM),
        ],
        out_specs=[
            pl.BlockSpec(memory_space=pltpu.VMEM),
            pl.BlockSpec(memory_space=pltpu.HBM),
        ],
        scratch_shapes=[
            pltpu.VMEM((3, 4, m_per, kp_max), jnp.float32),
            pltpu.VMEM((2, m_per, kp_max), jnp.float32),
            pltpu.VMEM((N_DEV, 8, 128), jnp.float32),
            pltpu.SemaphoreType.DMA((3, N_DEV)),
            pltpu.SemaphoreType.DMA((3, N_DEV)),
            pltpu.SemaphoreType.DMA((2,)),
            pltpu.SemaphoreType.DMA((N_DEV,)),
            pltpu.SemaphoreType.DMA((N_DEV,)),
        ],
        compiler_params=pltpu.CompilerParams(
            collective_id=0,
            vmem_limit_bytes=100 * 1024 * 1024,
        ),
    )(x, w_mat)
    return out
